# baseline (speedup 1.0000x reference)
import functools

import jax
import jax.numpy as jnp
from jax.experimental import pallas as pl
from jax.experimental.pallas import tpu as pltpu

_TAPS = tuple((dy, dx) for dy in (-1, 0, 1) for dx in (-1, 0, 1))
_FUT = 10


def _conv_body(x_ref, w1x_r, w1h_r, b1_r, w2_r, b2_r, w3_r, b3_r, w4_r, b4_r,
               wc_r, bc_r, o_ref, c1, c2, c3, c4, colE, colD,
               *, T, F, nf, H, W, S):
    N1 = H * W
    N = S * N1
    K = 9 * nf

    pos = jax.lax.broadcasted_iota(jnp.int32, (1, N), 1) % N1
    px, py = pos % W, pos // W
    masks = []
    for dy, dx in _TAPS:
        if dy == 0 and dx == 0:
            masks.append(None)
            continue
        ok = (px + dx >= 0) & (px + dx < W) & (py + dy >= 0) & (py + dy < H)
        masks.append(ok.astype(jnp.bfloat16))

    def im2col(img):
        imgb = img.astype(jnp.bfloat16)
        parts = []
        for (dy, dx), m in zip(_TAPS, masks):
            o = dy * W + dx
            r = imgb if o == 0 else pltpu.roll(imgb, shift=(-o) % N, axis=1)
            parts.append(r if m is None else r * m)
        return jnp.concatenate(parts, axis=0)

    def mm(w, col):
        return jnp.dot(w, col, preferred_element_type=jnp.float32)

    def gates(acc, c_prev):
        sig = 0.5 * jnp.tanh(0.5 * acc[:3 * nf]) + 0.5
        g = jnp.tanh(acc[3 * nf:])
        ig = sig[:nf] * g
        c_n = ig if c_prev is None else sig[nf:2 * nf] * c_prev + ig
        return c_n, sig[2 * nf:] * jnp.tanh(c_n)

    def xt(t):
        if S == 1:
            return x_ref[0, t]
        return jnp.concatenate([x_ref[s, t] for s in range(S)], axis=1)

    a1 = mm(w1x_r[...], im2col(xt(0))) + b1_r[...]
    c1n, h1 = gates(a1, None)
    c1[...] = c1n
    ch1 = im2col(h1)
    colE[0:K] = ch1
    a2 = mm(w2_r[:, :K], ch1) + b2_r[...]
    c2n, h2 = gates(a2, None)
    c2[...] = c2n
    colE[K:] = im2col(h2)

    def enc_body(t, carry):
        a1 = (mm(w1x_r[...], im2col(xt(t)))
              + mm(w1h_r[...], colE[0:K]) + b1_r[...])
        c1n, h1 = gates(a1, c1[...])
        c1[...] = c1n
        colE[0:K] = im2col(h1)
        a2 = mm(w2_r[...], colE[...]) + b2_r[...]
        c2n, h2 = gates(a2, c2[...])
        c2[...] = c2n
        colE[K:] = im2col(h2)
        return carry

    jax.lax.fori_loop(1, T, enc_body, 0)

    sub = jax.lax.broadcasted_iota(jnp.int32, (F, N1), 0)
    bc = bc_r[0, 0]

    def store_row(f, row, init):
        for s in range(S):
            part = row[:, s * N1:(s + 1) * N1]
            prev = 0.0 if init else o_ref[s]
            o_ref[s] = jnp.where(sub == f, part, prev)

    a3 = mm(w3_r[:, K:], colE[K:]) + b3_r[...]
    c3n, h3 = gates(a3, None)
    c3[...] = c3n
    ch3 = im2col(h3)
    colD[0:K] = ch3
    a4 = mm(w4_r[:, :K], ch3) + b4_r[...]
    c4n, h4 = gates(a4, None)
    c4[...] = c4n
    col4 = im2col(h4)
    colD[K:] = col4
    row = jax.nn.sigmoid(mm(wc_r[...], col4)[0:1] + bc)
    store_row(0, row, init=True)

    def dec_body(f, carry):
        a3 = mm(w3_r[...], colD[...]) + b3_r[...]
        c3n, h3 = gates(a3, c3[...])
        c3[...] = c3n
        colD[0:K] = im2col(h3)
        a4 = mm(w4_r[...], colD[...]) + b4_r[...]
        c4n, h4 = gates(a4, c4[...])
        c4[...] = c4n
        col4 = im2col(h4)
        colD[K:] = col4
        row = jax.nn.sigmoid(mm(wc_r[...], col4)[0:1] + bc)
        store_row(f, row, init=False)
        return carry

    jax.lax.fori_loop(1, F, dec_body, 0)


def _layout_w(w9, cin, cin_pad, nf):
    cout = w9.shape[-1]
    wx9 = w9[:, :cin, :]
    if cin_pad != cin:
        pad = jnp.zeros((9, cin_pad - cin, cout), w9.dtype)
        wx9 = jnp.concatenate([wx9, pad], axis=1)
    wx = jnp.transpose(wx9, (2, 0, 1)).reshape(cout, 9 * cin_pad)
    wh = jnp.transpose(w9[:, cin:, :], (2, 0, 1)).reshape(cout, 9 * nf)
    return wx.astype(jnp.bfloat16), wh.astype(jnp.bfloat16)


@jax.jit
def kernel(enc1_w, enc1_b, enc2_w, enc2_b, dec1_w, dec1_b, dec2_w, dec2_b,
           cnn_w, cnn_b, x):
    b, T, cin, H, W = x.shape
    nf = enc1_w.shape[-1] // 4
    F = _FUT
    N = H * W
    cin_pad = ((cin + 7) // 8) * 8
    K = 9 * nf
    S = 4 if b % 4 == 0 else (2 if b % 2 == 0 else 1)

    xb = x.astype(jnp.bfloat16).reshape(b, T, cin, N)
    xb = jnp.pad(xb, ((0, 0), (0, 0), (0, cin_pad - cin), (0, 0)))

    w1x, w1h = _layout_w(enc1_w, cin, cin_pad, nf)
    w2x, w2h = _layout_w(enc2_w, nf, nf, nf)
    w3x, w3h = _layout_w(dec1_w, nf, nf, nf)
    w4x, w4h = _layout_w(dec2_w, nf, nf, nf)
    w2 = jnp.concatenate([w2x, w2h], axis=1)
    w3 = jnp.concatenate([w3h, w3x], axis=1)
    w4 = jnp.concatenate([w4x, w4h], axis=1)
    wc_row = jnp.transpose(cnn_w, (2, 0, 1)).reshape(1, K)
    wc = jnp.pad(wc_row, ((0, 7), (0, 0))).astype(jnp.bfloat16)

    b1 = enc1_b.reshape(-1, 1)
    b2 = enc2_b.reshape(-1, 1)
    b3 = dec1_b.reshape(-1, 1)
    b4 = dec2_b.reshape(-1, 1)
    bc = cnn_b.reshape(1, 1)

    body = functools.partial(_conv_body, T=T, F=F, nf=nf, H=H, W=W, S=S)

    NS = S * N
    w_args = (w1x, w1h, b1, w2, b2, w3, b3, w4, b4, wc)
    in_specs = [pl.BlockSpec((S, T, cin_pad, N), lambda i: (i, 0, 0, 0))]
    in_specs += [pl.BlockSpec(w.shape, lambda i: (0, 0)) for w in w_args]
    in_specs += [pl.BlockSpec(memory_space=pltpu.MemorySpace.SMEM)]

    out = pl.pallas_call(
        body,
        out_shape=jax.ShapeDtypeStruct((b, F, N), jnp.float32),
        grid=(b // S,),
        in_specs=in_specs,
        out_specs=pl.BlockSpec((S, F, N), lambda i: (i, 0, 0)),
        scratch_shapes=[pltpu.VMEM((nf, NS), jnp.float32)] * 4
                     + [pltpu.VMEM((2 * K, NS), jnp.bfloat16)] * 2,
        compiler_params=pltpu.CompilerParams(
            dimension_semantics=("parallel",),
            vmem_limit_bytes=64 * 1024 * 1024),
    )(xb, *w_args, bc)

    return out.reshape(b, F, H, W)[:, None, :, :, :]


# speedup vs baseline: 1.0121x; 1.0121x over previous
import functools

import jax
import jax.numpy as jnp
from jax.experimental import pallas as pl
from jax.experimental.pallas import tpu as pltpu

_TAPS = tuple((dy, dx) for dy in (-1, 0, 1) for dx in (-1, 0, 1))
_FUT = 10


def _conv_body(x_ref, w1x_r, w1h_r, b1_r, w2_r, b2_r, w3_r, b3_r, w4_r, b4_r,
               wc_r, bc_r, o_ref, c1, c2, c3, c4, colE, colD,
               *, T, F, nf, H, W, S):
    N1 = H * W
    N = S * N1
    K = 9 * nf

    pos = jax.lax.broadcasted_iota(jnp.int32, (1, N), 1) % N1
    px, py = pos % W, pos // W
    masks = []
    for dy, dx in _TAPS:
        if dy == 0 and dx == 0:
            masks.append(None)
            continue
        ok = (px + dx >= 0) & (px + dx < W) & (py + dy >= 0) & (py + dy < H)
        masks.append(ok.astype(jnp.bfloat16))

    def im2col(img):
        imgb = img.astype(jnp.bfloat16)
        parts = []
        for (dy, dx), m in zip(_TAPS, masks):
            o = dy * W + dx
            r = imgb if o == 0 else pltpu.roll(imgb, shift=(-o) % N, axis=1)
            parts.append(r if m is None else r * m)
        return jnp.concatenate(parts, axis=0)

    def mm(w, col):
        return jnp.dot(w, col, preferred_element_type=jnp.float32)

    def gates(acc, c_prev):
        sig = jax.nn.sigmoid(acc[:3 * nf])
        g = jnp.tanh(acc[3 * nf:])
        ig = sig[:nf] * g
        c_n = ig if c_prev is None else sig[nf:2 * nf] * c_prev + ig
        return c_n, sig[2 * nf:] * jnp.tanh(c_n)

    def xt(t):
        if S == 1:
            return x_ref[0, t]
        return jnp.concatenate([x_ref[s, t] for s in range(S)], axis=1)

    a1 = mm(w1x_r[...], im2col(xt(0))) + b1_r[...]
    c1n, h1 = gates(a1, None)
    c1[...] = c1n
    ch1 = im2col(h1)
    colE[0:K] = ch1
    a2 = mm(w2_r[:, :K], ch1) + b2_r[...]
    c2n, h2 = gates(a2, None)
    c2[...] = c2n
    colE[K:] = im2col(h2)

    def enc_body(t, carry):
        a1 = (mm(w1x_r[...], im2col(xt(t)))
              + mm(w1h_r[...], colE[0:K]) + b1_r[...])
        c1n, h1 = gates(a1, c1[...])
        c1[...] = c1n
        colE[0:K] = im2col(h1)
        a2 = mm(w2_r[...], colE[...]) + b2_r[...]
        c2n, h2 = gates(a2, c2[...])
        c2[...] = c2n
        colE[K:] = im2col(h2)
        return carry

    jax.lax.fori_loop(1, T, enc_body, 0)

    sub = jax.lax.broadcasted_iota(jnp.int32, (F, N1), 0)
    bc = bc_r[0, 0]

    def store_row(f, row, init):
        for s in range(S):
            part = row[:, s * N1:(s + 1) * N1]
            prev = 0.0 if init else o_ref[s]
            o_ref[s] = jnp.where(sub == f, part, prev)

    a3 = mm(w3_r[:, K:], colE[K:]) + b3_r[...]
    c3n, h3 = gates(a3, None)
    c3[...] = c3n
    ch3 = im2col(h3)
    colD[0:K] = ch3
    a4 = mm(w4_r[:, :K], ch3) + b4_r[...]
    c4n, h4 = gates(a4, None)
    c4[...] = c4n
    col4 = im2col(h4)
    colD[K:] = col4
    row = jax.nn.sigmoid(mm(wc_r[...], col4)[0:1] + bc)
    store_row(0, row, init=True)

    def dec_body(f, carry):
        a3 = mm(w3_r[...], colD[...]) + b3_r[...]
        c3n, h3 = gates(a3, c3[...])
        c3[...] = c3n
        colD[0:K] = im2col(h3)
        a4 = mm(w4_r[...], colD[...]) + b4_r[...]
        c4n, h4 = gates(a4, c4[...])
        c4[...] = c4n
        col4 = im2col(h4)
        colD[K:] = col4
        row = jax.nn.sigmoid(mm(wc_r[...], col4)[0:1] + bc)
        store_row(f, row, init=False)
        return carry

    jax.lax.fori_loop(1, F, dec_body, 0)


def _layout_w(w9, cin, cin_pad, nf):
    cout = w9.shape[-1]
    wx9 = w9[:, :cin, :]
    if cin_pad != cin:
        pad = jnp.zeros((9, cin_pad - cin, cout), w9.dtype)
        wx9 = jnp.concatenate([wx9, pad], axis=1)
    wx = jnp.transpose(wx9, (2, 0, 1)).reshape(cout, 9 * cin_pad)
    wh = jnp.transpose(w9[:, cin:, :], (2, 0, 1)).reshape(cout, 9 * nf)
    return wx.astype(jnp.bfloat16), wh.astype(jnp.bfloat16)


@jax.jit
def kernel(enc1_w, enc1_b, enc2_w, enc2_b, dec1_w, dec1_b, dec2_w, dec2_b,
           cnn_w, cnn_b, x):
    b, T, cin, H, W = x.shape
    nf = enc1_w.shape[-1] // 4
    F = _FUT
    N = H * W
    cin_pad = ((cin + 7) // 8) * 8
    K = 9 * nf
    S = 4 if b % 4 == 0 else (2 if b % 2 == 0 else 1)

    xb = x.astype(jnp.bfloat16).reshape(b, T, cin, N)
    xb = jnp.pad(xb, ((0, 0), (0, 0), (0, cin_pad - cin), (0, 0)))

    w1x, w1h = _layout_w(enc1_w, cin, cin_pad, nf)
    w2x, w2h = _layout_w(enc2_w, nf, nf, nf)
    w3x, w3h = _layout_w(dec1_w, nf, nf, nf)
    w4x, w4h = _layout_w(dec2_w, nf, nf, nf)
    w2 = jnp.concatenate([w2x, w2h], axis=1)
    w3 = jnp.concatenate([w3h, w3x], axis=1)
    w4 = jnp.concatenate([w4x, w4h], axis=1)
    wc_row = jnp.transpose(cnn_w, (2, 0, 1)).reshape(1, K)
    wc = jnp.pad(wc_row, ((0, 7), (0, 0))).astype(jnp.bfloat16)

    b1 = enc1_b.reshape(-1, 1)
    b2 = enc2_b.reshape(-1, 1)
    b3 = dec1_b.reshape(-1, 1)
    b4 = dec2_b.reshape(-1, 1)
    bc = cnn_b.reshape(1, 1)

    body = functools.partial(_conv_body, T=T, F=F, nf=nf, H=H, W=W, S=S)

    NS = S * N
    w_args = (w1x, w1h, b1, w2, b2, w3, b3, w4, b4, wc)
    in_specs = [pl.BlockSpec((S, T, cin_pad, N), lambda i: (i, 0, 0, 0))]
    in_specs += [pl.BlockSpec(w.shape, lambda i: (0, 0)) for w in w_args]
    in_specs += [pl.BlockSpec(memory_space=pltpu.MemorySpace.SMEM)]

    out = pl.pallas_call(
        body,
        out_shape=jax.ShapeDtypeStruct((b, F, N), jnp.float32),
        grid=(b // S,),
        in_specs=in_specs,
        out_specs=pl.BlockSpec((S, F, N), lambda i: (i, 0, 0)),
        scratch_shapes=[pltpu.VMEM((nf, NS), jnp.float32)] * 4
                     + [pltpu.VMEM((2 * K, NS), jnp.bfloat16)] * 2,
        compiler_params=pltpu.CompilerParams(
            dimension_semantics=("parallel",),
            vmem_limit_bytes=64 * 1024 * 1024),
    )(xb, *w_args, bc)

    return out.reshape(b, F, H, W)[:, None, :, :, :]


# speedup vs baseline: 1.0272x; 1.0149x over previous
import functools

import jax
import jax.numpy as jnp
from jax.experimental import pallas as pl
from jax.experimental.pallas import tpu as pltpu

_TAPS = tuple((dy, dx) for dy in (-1, 0, 1) for dx in (-1, 0, 1))
_FUT = 10


def _conv_body(x_ref, w1x_r, w1h_r, b1_r, w2_r, b2_r, w3_r, b3_r, w4_r, b4_r,
               wc_r, bc_r, o_ref, *scr, T, F, nf, H, W, Sg, G):
    N1 = H * W
    N = Sg * N1
    K = 9 * nf
    groups = [scr[g * 6:(g + 1) * 6] for g in range(G)]

    pos = jax.lax.broadcasted_iota(jnp.int32, (1, N), 1) % N1
    px, py = pos % W, pos // W
    masks = []
    for dy, dx in _TAPS:
        if dy == 0 and dx == 0:
            masks.append(None)
            continue
        ok = (px + dx >= 0) & (px + dx < W) & (py + dy >= 0) & (py + dy < H)
        masks.append(ok.astype(jnp.bfloat16))

    def im2col(img):
        imgb = img.astype(jnp.bfloat16)
        parts = []
        for (dy, dx), m in zip(_TAPS, masks):
            o = dy * W + dx
            r = imgb if o == 0 else pltpu.roll(imgb, shift=(-o) % N, axis=1)
            parts.append(r if m is None else r * m)
        return jnp.concatenate(parts, axis=0)

    def mm(w, col):
        return jnp.dot(w, col, preferred_element_type=jnp.float32)

    def gates(acc, c_prev):
        sig = jax.nn.sigmoid(acc[:3 * nf])
        g = jnp.tanh(acc[3 * nf:])
        ig = sig[:nf] * g
        c_n = ig if c_prev is None else sig[nf:2 * nf] * c_prev + ig
        return c_n, sig[2 * nf:] * jnp.tanh(c_n)

    def xt(g, t):
        if Sg == 1:
            return x_ref[g, t]
        return jnp.concatenate(
            [x_ref[g * Sg + s, t] for s in range(Sg)], axis=1)

    def enc_step(g, t, first):
        c1, c2, _, _, colE, _ = groups[g]
        a1 = mm(w1x_r[...], im2col(xt(g, t))) + b1_r[...]
        if first:
            c1n, h1 = gates(a1, None)
        else:
            c1n, h1 = gates(a1 + mm(w1h_r[...], colE[0:K]), c1[...])
        c1[...] = c1n
        ch1 = im2col(h1)
        colE[0:K] = ch1
        if first:
            a2 = mm(w2_r[:, :K], ch1) + b2_r[...]
            c2n, h2 = gates(a2, None)
        else:
            a2 = mm(w2_r[...], colE[...]) + b2_r[...]
            c2n, h2 = gates(a2, c2[...])
        c2[...] = c2n
        colE[K:] = im2col(h2)

    sub = jax.lax.broadcasted_iota(jnp.int32, (F, N1), 0)
    bc = bc_r[0, 0]

    def store_row(g, f, row, init):
        for s in range(Sg):
            part = row[:, s * N1:(s + 1) * N1]
            prev = 0.0 if init else o_ref[g * Sg + s]
            o_ref[g * Sg + s] = jnp.where(sub == f, part, prev)

    def dec_step(g, f, first):
        _, _, c3, c4, colE, colD = groups[g]
        if first:
            a3 = mm(w3_r[:, K:], colE[K:]) + b3_r[...]
            c3n, h3 = gates(a3, None)
        else:
            a3 = mm(w3_r[...], colD[...]) + b3_r[...]
            c3n, h3 = gates(a3, c3[...])
        c3[...] = c3n
        ch3 = im2col(h3)
        colD[0:K] = ch3
        if first:
            a4 = mm(w4_r[:, :K], ch3) + b4_r[...]
            c4n, h4 = gates(a4, None)
        else:
            a4 = mm(w4_r[...], colD[...]) + b4_r[...]
            c4n, h4 = gates(a4, c4[...])
        c4[...] = c4n
        col4 = im2col(h4)
        colD[K:] = col4
        row = jax.nn.sigmoid(mm(wc_r[...], col4)[0:1] + bc)
        store_row(g, f, row, init=first)

    for g in range(G):
        enc_step(g, 0, first=True)

    def enc_body(t, carry):
        for g in range(G):
            enc_step(g, t, first=False)
        return carry

    jax.lax.fori_loop(1, T, enc_body, 0)

    for g in range(G):
        dec_step(g, 0, first=True)

    def dec_body(f, carry):
        for g in range(G):
            dec_step(g, f, first=False)
        return carry

    jax.lax.fori_loop(1, F, dec_body, 0)


def _layout_w(w9, cin, cin_pad, nf):
    cout = w9.shape[-1]
    wx9 = w9[:, :cin, :]
    if cin_pad != cin:
        pad = jnp.zeros((9, cin_pad - cin, cout), w9.dtype)
        wx9 = jnp.concatenate([wx9, pad], axis=1)
    wx = jnp.transpose(wx9, (2, 0, 1)).reshape(cout, 9 * cin_pad)
    wh = jnp.transpose(w9[:, cin:, :], (2, 0, 1)).reshape(cout, 9 * nf)
    return wx.astype(jnp.bfloat16), wh.astype(jnp.bfloat16)


@jax.jit
def kernel(enc1_w, enc1_b, enc2_w, enc2_b, dec1_w, dec1_b, dec2_w, dec2_b,
           cnn_w, cnn_b, x):
    b, T, cin, H, W = x.shape
    nf = enc1_w.shape[-1] // 4
    F = _FUT
    N = H * W
    cin_pad = ((cin + 7) // 8) * 8
    K = 9 * nf
    if b % 4 == 0:
        Sg, G = 2, 2
    elif b % 2 == 0:
        Sg, G = 2, 1
    else:
        Sg, G = 1, 1
    S = Sg * G

    xb = x.astype(jnp.bfloat16).reshape(b, T, cin, N)
    xb = jnp.pad(xb, ((0, 0), (0, 0), (0, cin_pad - cin), (0, 0)))

    w1x, w1h = _layout_w(enc1_w, cin, cin_pad, nf)
    w2x, w2h = _layout_w(enc2_w, nf, nf, nf)
    w3x, w3h = _layout_w(dec1_w, nf, nf, nf)
    w4x, w4h = _layout_w(dec2_w, nf, nf, nf)
    w2 = jnp.concatenate([w2x, w2h], axis=1)
    w3 = jnp.concatenate([w3h, w3x], axis=1)
    w4 = jnp.concatenate([w4x, w4h], axis=1)
    wc_row = jnp.transpose(cnn_w, (2, 0, 1)).reshape(1, K)
    wc = jnp.pad(wc_row, ((0, 7), (0, 0))).astype(jnp.bfloat16)

    b1 = enc1_b.reshape(-1, 1)
    b2 = enc2_b.reshape(-1, 1)
    b3 = dec1_b.reshape(-1, 1)
    b4 = dec2_b.reshape(-1, 1)
    bc = cnn_b.reshape(1, 1)

    body = functools.partial(_conv_body, T=T, F=F, nf=nf, H=H, W=W, Sg=Sg, G=G)

    NS = Sg * N
    w_args = (w1x, w1h, b1, w2, b2, w3, b3, w4, b4, wc)
    in_specs = [pl.BlockSpec((S, T, cin_pad, N), lambda i: (i, 0, 0, 0))]
    in_specs += [pl.BlockSpec(w.shape, lambda i: (0, 0)) for w in w_args]
    in_specs += [pl.BlockSpec(memory_space=pltpu.MemorySpace.SMEM)]

    group_scratch = ([pltpu.VMEM((nf, NS), jnp.float32)] * 4
                     + [pltpu.VMEM((2 * K, NS), jnp.bfloat16)] * 2)

    out = pl.pallas_call(
        body,
        out_shape=jax.ShapeDtypeStruct((b, F, N), jnp.float32),
        grid=(b // S,),
        in_specs=in_specs,
        out_specs=pl.BlockSpec((S, F, N), lambda i: (i, 0, 0)),
        scratch_shapes=group_scratch * G,
        compiler_params=pltpu.CompilerParams(
            dimension_semantics=("parallel",),
            vmem_limit_bytes=64 * 1024 * 1024),
    )(xb, *w_args, bc)

    return out.reshape(b, F, H, W)[:, None, :, :, :]
